# Initial kernel scaffold; baseline (speedup 1.0000x reference)
import jax
import jax.numpy as jnp
from jax.experimental import pallas as pl


def kernel(drug_x, edge_index, batch, protein_seq, drug_ids, prot_ids, affinity_x, affinity_edge_index, affinity_adj, Wd1, bd1, Wd2, bd2, emb, cw1, cb1, cw2, cb2, Wg1, bg1, Wg2, bg2, pw1, pb1, pw2, pb2):
    raise NotImplementedError("write your pallas kernel here")


# speedup vs baseline: 2.4473x; 2.4473x over previous
import functools

import jax
import jax.numpy as jnp
from jax import lax
from jax.experimental import pallas as pl
from jax.experimental.pallas import tpu as pltpu

EMB = 128
B = 1024
TARGET_LEN = 128
N_ATOM = 32768
NUM_DRUG = 2000
N_AFF = 10000
NAFF_PAD = 10240
TEMP = 0.5


def _leaky(x):
    return jnp.where(x >= 0, x, 0.01 * x)


def _mm_scale_body(x_ref, w_ref, deg_ref, o_ref):
    dinv = lax.rsqrt(deg_ref[...] + 1.0)
    o_ref[...] = dinv * jnp.dot(x_ref[...], w_ref[...],
                                preferred_element_type=jnp.float32)


def _mm_scale(x, w, deg, bm=2048):
    n = x.shape[0]
    return pl.pallas_call(
        _mm_scale_body,
        grid=(n // bm,),
        in_specs=[pl.BlockSpec((bm, EMB), lambda i: (i, 0)),
                  pl.BlockSpec((EMB, EMB), lambda i: (0, 0)),
                  pl.BlockSpec((bm, 1), lambda i: (i, 0))],
        out_specs=pl.BlockSpec((bm, EMB), lambda i: (i, 0)),
        out_shape=jax.ShapeDtypeStruct((n, EMB), jnp.float32),
    )(x, w, deg)


def _combine_mm_body(msg_ref, hs_ref, deg_ref, b_ref, w_ref, o_ref):
    dinv = lax.rsqrt(deg_ref[...] + 1.0)
    g = _leaky(dinv * (msg_ref[...] + hs_ref[...]) + b_ref[...])
    o_ref[...] = dinv * jnp.dot(g, w_ref[...],
                                preferred_element_type=jnp.float32)


def _combine_mm(msg, hs, deg, b, w, bm=2048):
    n = msg.shape[0]
    return pl.pallas_call(
        _combine_mm_body,
        grid=(n // bm,),
        in_specs=[pl.BlockSpec((bm, EMB), lambda i: (i, 0)),
                  pl.BlockSpec((bm, EMB), lambda i: (i, 0)),
                  pl.BlockSpec((bm, 1), lambda i: (i, 0)),
                  pl.BlockSpec((1, EMB), lambda i: (0, 0)),
                  pl.BlockSpec((EMB, EMB), lambda i: (0, 0))],
        out_specs=pl.BlockSpec((bm, EMB), lambda i: (i, 0)),
        out_shape=jax.ShapeDtypeStruct((n, EMB), jnp.float32),
    )(msg, hs, deg, b, w)


def _combine_body(msg_ref, hs_ref, deg_ref, b_ref, o_ref):
    dinv = lax.rsqrt(deg_ref[...] + 1.0)
    o_ref[...] = _leaky(dinv * (msg_ref[...] + hs_ref[...]) + b_ref[...])


def _combine(msg, hs, deg, b, bm=2048):
    n = msg.shape[0]
    return pl.pallas_call(
        _combine_body,
        grid=(n // bm,),
        in_specs=[pl.BlockSpec((bm, EMB), lambda i: (i, 0)),
                  pl.BlockSpec((bm, EMB), lambda i: (i, 0)),
                  pl.BlockSpec((bm, 1), lambda i: (i, 0)),
                  pl.BlockSpec((1, EMB), lambda i: (0, 0))],
        out_specs=pl.BlockSpec((bm, EMB), lambda i: (i, 0)),
        out_shape=jax.ShapeDtypeStruct((n, EMB), jnp.float32),
    )(msg, hs, deg, b)


_NB = 32


def _prot_body(seq_ref, emb_ref, w1_ref, b1_ref, w2_ref, b2_ref, o_ref):
    R = _NB * TARGET_LEN
    seq = seq_ref[...]
    cols = lax.broadcasted_iota(jnp.int32, (R, 32), 1)
    onehot = jnp.where(seq == cols, 1.0, 0.0)
    x = jnp.dot(onehot, emb_ref[...], preferred_element_type=jnp.float32)
    pos = lax.broadcasted_iota(jnp.int32, (R, 1), 0) % TARGET_LEN
    zrow = jnp.zeros((1, EMB), jnp.float32)

    def block(xx, w_ref, b_ref):
        down = jnp.concatenate([zrow, xx[:-1, :]], axis=0)
        up = jnp.concatenate([xx[1:, :], zrow], axis=0)
        down = jnp.where(pos == 0, 0.0, down)
        up = jnp.where(pos == TARGET_LEN - 1, 0.0, up)
        y = (jnp.dot(down, w_ref[0], preferred_element_type=jnp.float32)
             + jnp.dot(xx, w_ref[1], preferred_element_type=jnp.float32)
             + jnp.dot(up, w_ref[2], preferred_element_type=jnp.float32)
             + b_ref[...])
        y = y * (1.0 / jnp.sqrt(1.0 + 1e-05))
        return _leaky(y) + xx

    x = block(x, w1_ref, b1_ref)
    x = block(x, w2_ref, b2_ref)
    rr = lax.broadcasted_iota(jnp.int32, (_NB, R), 1)
    bb = lax.broadcasted_iota(jnp.int32, (_NB, R), 0)
    sel = jnp.where(rr // TARGET_LEN == bb, 1.0 / TARGET_LEN, 0.0)
    o_ref[...] = jnp.dot(sel, x, preferred_element_type=jnp.float32)


def _prot_head(seq_flat, emb_pad, w1s, b1, w2s, b2):
    R = _NB * TARGET_LEN
    return pl.pallas_call(
        _prot_body,
        grid=(B // _NB,),
        in_specs=[pl.BlockSpec((R, 1), lambda i: (i, 0)),
                  pl.BlockSpec((32, EMB), lambda i: (0, 0)),
                  pl.BlockSpec((3, EMB, EMB), lambda i: (0, 0, 0)),
                  pl.BlockSpec((1, EMB), lambda i: (0, 0)),
                  pl.BlockSpec((3, EMB, EMB), lambda i: (0, 0, 0)),
                  pl.BlockSpec((1, EMB), lambda i: (0, 0))],
        out_specs=pl.BlockSpec((_NB, EMB), lambda i: (i, 0)),
        out_shape=jax.ShapeDtypeStruct((B, EMB), jnp.float32),
    )(seq_flat, emb_pad, w1s, b1, w2s, b2)


def _segsum_body(h_ref, b_ref, o_ref, c_ref):
    i = pl.program_id(0)
    bm = h_ref.shape[0]
    cols = lax.broadcasted_iota(jnp.int32, (bm, B), 1)
    m = jnp.where(b_ref[...] == cols, 1.0, 0.0)
    s = lax.dot_general(m, h_ref[...], (((0,), (0,)), ((), ())),
                        preferred_element_type=jnp.float32)

    @pl.when(i == 0)
    def _():
        o_ref[...] = jnp.zeros_like(o_ref)
        c_ref[...] = jnp.zeros_like(c_ref)

    o_ref[...] += s
    c_ref[...] += jnp.sum(m, axis=0, keepdims=True)


def _segsum(h, batch2d, bm=2048):
    n = h.shape[0]
    return pl.pallas_call(
        _segsum_body,
        grid=(n // bm,),
        in_specs=[pl.BlockSpec((bm, EMB), lambda i: (i, 0)),
                  pl.BlockSpec((bm, 1), lambda i: (i, 0))],
        out_specs=[pl.BlockSpec((B, EMB), lambda i: (0, 0)),
                   pl.BlockSpec((1, B), lambda i: (0, 0))],
        out_shape=[jax.ShapeDtypeStruct((B, EMB), jnp.float32),
                   jax.ShapeDtypeStruct((1, B), jnp.float32)],
    )(h, batch2d)


def _rownorm_body(x_ref, o_ref):
    x = x_ref[...]
    nrm = jnp.sqrt(jnp.sum(x * x, axis=1, keepdims=True))
    o_ref[...] = x / jnp.maximum(nrm, 1e-12)


def _rownorm(x):
    return pl.pallas_call(
        _rownorm_body,
        grid=(1,),
        in_specs=[pl.BlockSpec((B, EMB), lambda i: (0, 0))],
        out_specs=pl.BlockSpec((B, EMB), lambda i: (0, 0)),
        out_shape=jax.ShapeDtypeStruct((B, EMB), jnp.float32),
    )(x)


_NROW = 128


def _simclr_body(reps_ref, a_ref, p_ref, o_ref):
    i = pl.program_id(0)
    n2 = 2 * B
    a = a_ref[...]
    s = lax.dot_general(a, reps_ref[...], (((1,), (1,)), ((), ())),
                        preferred_element_type=jnp.float32) / TEMP
    rows = lax.broadcasted_iota(jnp.int32, (_NROW, n2), 0) + i * _NROW
    cols = lax.broadcasted_iota(jnp.int32, (_NROW, n2), 1)
    s = jnp.where(rows == cols, -1e30, s)
    m = jnp.max(s, axis=1, keepdims=True)
    lse = m + jnp.log(jnp.sum(jnp.exp(s - m), axis=1, keepdims=True))
    d = jnp.sum(a * p_ref[...], axis=1, keepdims=True) / TEMP

    @pl.when(i == 0)
    def _():
        o_ref[...] = jnp.zeros_like(o_ref)

    contrib = jnp.sum(lse - d, axis=0, keepdims=True) / n2
    o_ref[...] += contrib


def _simclr(repsn):
    nblk = 2 * B // _NROW
    return pl.pallas_call(
        _simclr_body,
        grid=(nblk,),
        in_specs=[pl.BlockSpec((2 * B, EMB), lambda i: (0, 0)),
                  pl.BlockSpec((_NROW, EMB), lambda i: (i, 0)),
                  pl.BlockSpec((_NROW, EMB),
                               lambda i: (lax.rem(i + nblk // 2, nblk), 0))],
        out_specs=pl.BlockSpec((1, 1), lambda i: (0, 0)),
        out_shape=jax.ShapeDtypeStruct((1, 1), jnp.float32),
    )(repsn, repsn, repsn)


def _mlp_body(seg_ref, cnt_ref, dg_ref, pr_ref, pg_ref,
              w1_ref, b1_ref, w2_ref, b2_ref, o_ref):
    rep = seg_ref[...] / jnp.maximum(cnt_ref[...], 1.0)
    hid = (jnp.dot(rep, w1_ref[pl.ds(0, EMB), :],
                   preferred_element_type=jnp.float32)
           + jnp.dot(dg_ref[...], w1_ref[pl.ds(EMB, EMB), :],
                     preferred_element_type=jnp.float32)
           + jnp.dot(pr_ref[...], w1_ref[pl.ds(2 * EMB, EMB), :],
                     preferred_element_type=jnp.float32)
           + jnp.dot(pg_ref[...], w1_ref[pl.ds(3 * EMB, EMB), :],
                     preferred_element_type=jnp.float32)
           + b1_ref[...])
    hid = _leaky(hid)
    o_ref[...] = lax.dot_general(hid, w2_ref[...], (((1,), (0,)), ((), ())),
                                 preferred_element_type=jnp.float32) + b2_ref[...]


def _mlp(seg, cnt, dg, pr, pg, w1, b1, w2, b2):
    return pl.pallas_call(
        _mlp_body,
        grid=(1,),
        in_specs=[pl.BlockSpec((B, EMB), lambda i: (0, 0)),
                  pl.BlockSpec((B, 1), lambda i: (0, 0)),
                  pl.BlockSpec((B, EMB), lambda i: (0, 0)),
                  pl.BlockSpec((B, EMB), lambda i: (0, 0)),
                  pl.BlockSpec((B, EMB), lambda i: (0, 0)),
                  pl.BlockSpec((4 * EMB, EMB), lambda i: (0, 0)),
                  pl.BlockSpec((1, EMB), lambda i: (0, 0)),
                  pl.BlockSpec((EMB, 1), lambda i: (0, 0)),
                  pl.BlockSpec((1, 1), lambda i: (0, 0))],
        out_specs=pl.BlockSpec((B, 1), lambda i: (0, 0)),
        out_shape=jax.ShapeDtypeStruct((B, 1), jnp.float32),
    )(seg, cnt, dg, pr, pg, w1, b1, w2, b2)


def _scatter_rows(hs, src, dst, ew, nout):
    return jnp.zeros((nout, EMB), jnp.float32).at[dst].add(
        ew[:, None] * hs[src])


def _hist(idx, w, nout):
    return jnp.zeros((nout,), jnp.float32).at[idx].add(w)


def kernel(drug_x, edge_index, batch, protein_seq, drug_ids, prot_ids,
           affinity_x, affinity_edge_index, affinity_adj,
           Wd1, bd1, Wd2, bd2, emb, cw1, cb1, cw2, cb2,
           Wg1, bg1, Wg2, bg2, pw1, pb1, pw2, pb2):
    src = edge_index[0]
    dst = edge_index[1]
    gsrc = affinity_edge_index[0]
    gdst = affinity_edge_index[1]

    degd = _hist(dst, jnp.ones_like(dst, jnp.float32), N_ATOM)[:, None]
    hs1 = _mm_scale(drug_x, Wd1, degd)
    ones_e = jnp.ones((src.shape[0],), jnp.float32)
    msg1 = _scatter_rows(hs1, src, dst, ones_e, N_ATOM)
    hs2 = _combine_mm(msg1, hs1, degd, bd1[None, :], Wd2)
    msg2 = _scatter_rows(hs2, src, dst, ones_e, N_ATOM)
    g2d = _combine(msg2, hs2, degd, bd2[None, :])
    segsum, cnt = _segsum(g2d, batch[:, None])
    cnt_col = cnt.reshape(B, 1)

    emb_pad = jnp.concatenate([emb, jnp.zeros((6, EMB), jnp.float32)], axis=0)
    w1s = jnp.stack([cw1[:, :, k].T for k in range(3)], axis=0)
    w2s = jnp.stack([cw2[:, :, k].T for k in range(3)], axis=0)
    prot_rep = _prot_head(protein_seq.reshape(-1, 1), emb_pad,
                          w1s, cb1[None, :], w2s, cb2[None, :])

    ew = affinity_adj[gsrc, gdst]
    dega = _hist(gdst, ew, NAFF_PAD)[:, None]
    ax_pad = jnp.concatenate(
        [affinity_x, jnp.zeros((NAFF_PAD - N_AFF, EMB), jnp.float32)], axis=0)
    ga1 = _mm_scale(ax_pad, Wg1, dega)
    msga1 = _scatter_rows(ga1, gsrc, gdst, ew, NAFF_PAD)
    ga2 = _combine_mm(msga1, ga1, dega, bg1[None, :], Wg2)
    msga2 = _scatter_rows(ga2, gsrc, gdst, ew, NAFF_PAD)
    g2a = _combine(msga2, ga2, dega, bg2[None, :])

    drug_g = g2a[drug_ids]
    prot_g = g2a[prot_ids + NUM_DRUG]

    z1n = _rownorm(segsum)
    z2n = _rownorm(drug_g)
    loss_d = _simclr(jnp.concatenate([z1n, z2n], axis=0))[0, 0]
    p1n = _rownorm(prot_rep)
    p2n = _rownorm(prot_g)
    loss_p = _simclr(jnp.concatenate([p1n, p2n], axis=0))[0, 0]

    y = _mlp(segsum, cnt_col, drug_g, prot_rep, prot_g,
             pw1, pb1[None, :], pw2, pb2[None, :])
    return (y[:, 0], loss_d, loss_p)


# speedup vs baseline: 5.2192x; 2.1327x over previous
import functools

import jax
import jax.numpy as jnp
from jax import lax
from jax.experimental import pallas as pl
from jax.experimental.pallas import tpu as pltpu
from jax.experimental.pallas import tpu_sc as plsc

_NC = 2
_NS = 16
_CH = 128

EMB = 128
B = 1024
TARGET_LEN = 128
N_ATOM = 32768
NUM_DRUG = 2000
N_AFF = 10000
NAFF_PAD = 10240
TEMP = 0.5


def _leaky(x):
    return jnp.where(x >= 0, x, 0.01 * x)


def _mm_scale_body(x_ref, w_ref, deg_ref, o_ref):
    dinv = lax.rsqrt(deg_ref[...] + 1.0)
    o_ref[...] = dinv * jnp.dot(x_ref[...], w_ref[...],
                                preferred_element_type=jnp.float32)


def _mm_scale(x, w, deg, bm=2048):
    n = x.shape[0]
    return pl.pallas_call(
        _mm_scale_body,
        grid=(n // bm,),
        in_specs=[pl.BlockSpec((bm, EMB), lambda i: (i, 0)),
                  pl.BlockSpec((EMB, EMB), lambda i: (0, 0)),
                  pl.BlockSpec((bm, 1), lambda i: (i, 0))],
        out_specs=pl.BlockSpec((bm, EMB), lambda i: (i, 0)),
        out_shape=jax.ShapeDtypeStruct((n, EMB), jnp.float32),
    )(x, w, deg)


def _combine_mm_body(msg_ref, hs_ref, deg_ref, b_ref, w_ref, o_ref):
    dinv = lax.rsqrt(deg_ref[...] + 1.0)
    g = _leaky(dinv * (msg_ref[...] + hs_ref[...]) + b_ref[...])
    o_ref[...] = dinv * jnp.dot(g, w_ref[...],
                                preferred_element_type=jnp.float32)


def _combine_mm(msg, hs, deg, b, w, bm=2048):
    n = msg.shape[0]
    return pl.pallas_call(
        _combine_mm_body,
        grid=(n // bm,),
        in_specs=[pl.BlockSpec((bm, EMB), lambda i: (i, 0)),
                  pl.BlockSpec((bm, EMB), lambda i: (i, 0)),
                  pl.BlockSpec((bm, 1), lambda i: (i, 0)),
                  pl.BlockSpec((1, EMB), lambda i: (0, 0)),
                  pl.BlockSpec((EMB, EMB), lambda i: (0, 0))],
        out_specs=pl.BlockSpec((bm, EMB), lambda i: (i, 0)),
        out_shape=jax.ShapeDtypeStruct((n, EMB), jnp.float32),
    )(msg, hs, deg, b, w)


def _combine_body(msg_ref, hs_ref, deg_ref, b_ref, o_ref):
    dinv = lax.rsqrt(deg_ref[...] + 1.0)
    o_ref[...] = _leaky(dinv * (msg_ref[...] + hs_ref[...]) + b_ref[...])


def _combine(msg, hs, deg, b, bm=2048):
    n = msg.shape[0]
    return pl.pallas_call(
        _combine_body,
        grid=(n // bm,),
        in_specs=[pl.BlockSpec((bm, EMB), lambda i: (i, 0)),
                  pl.BlockSpec((bm, EMB), lambda i: (i, 0)),
                  pl.BlockSpec((bm, 1), lambda i: (i, 0)),
                  pl.BlockSpec((1, EMB), lambda i: (0, 0))],
        out_specs=pl.BlockSpec((bm, EMB), lambda i: (i, 0)),
        out_shape=jax.ShapeDtypeStruct((n, EMB), jnp.float32),
    )(msg, hs, deg, b)


_NB = 32


def _prot_body(seq_ref, emb_ref, w1_ref, b1_ref, w2_ref, b2_ref, o_ref):
    R = _NB * TARGET_LEN
    seq = seq_ref[...]
    cols = lax.broadcasted_iota(jnp.int32, (R, 32), 1)
    onehot = jnp.where(seq == cols, 1.0, 0.0)
    x = jnp.dot(onehot, emb_ref[...], preferred_element_type=jnp.float32)
    pos = lax.broadcasted_iota(jnp.int32, (R, 1), 0) % TARGET_LEN
    zrow = jnp.zeros((1, EMB), jnp.float32)

    def block(xx, w_ref, b_ref):
        down = jnp.concatenate([zrow, xx[:-1, :]], axis=0)
        up = jnp.concatenate([xx[1:, :], zrow], axis=0)
        down = jnp.where(pos == 0, 0.0, down)
        up = jnp.where(pos == TARGET_LEN - 1, 0.0, up)
        y = (jnp.dot(down, w_ref[0], preferred_element_type=jnp.float32)
             + jnp.dot(xx, w_ref[1], preferred_element_type=jnp.float32)
             + jnp.dot(up, w_ref[2], preferred_element_type=jnp.float32)
             + b_ref[...])
        y = y * (1.0 / jnp.sqrt(1.0 + 1e-05))
        return _leaky(y) + xx

    x = block(x, w1_ref, b1_ref)
    x = block(x, w2_ref, b2_ref)
    rr = lax.broadcasted_iota(jnp.int32, (_NB, R), 1)
    bb = lax.broadcasted_iota(jnp.int32, (_NB, R), 0)
    sel = jnp.where(rr // TARGET_LEN == bb, 1.0 / TARGET_LEN, 0.0)
    o_ref[...] = jnp.dot(sel, x, preferred_element_type=jnp.float32)


def _prot_head(seq_flat, emb_pad, w1s, b1, w2s, b2):
    R = _NB * TARGET_LEN
    return pl.pallas_call(
        _prot_body,
        grid=(B // _NB,),
        in_specs=[pl.BlockSpec((R, 1), lambda i: (i, 0)),
                  pl.BlockSpec((32, EMB), lambda i: (0, 0)),
                  pl.BlockSpec((3, EMB, EMB), lambda i: (0, 0, 0)),
                  pl.BlockSpec((1, EMB), lambda i: (0, 0)),
                  pl.BlockSpec((3, EMB, EMB), lambda i: (0, 0, 0)),
                  pl.BlockSpec((1, EMB), lambda i: (0, 0))],
        out_specs=pl.BlockSpec((_NB, EMB), lambda i: (i, 0)),
        out_shape=jax.ShapeDtypeStruct((B, EMB), jnp.float32),
    )(seq_flat, emb_pad, w1s, b1, w2s, b2)


def _segsum_body(h_ref, b_ref, o_ref, c_ref):
    i = pl.program_id(0)
    bm = h_ref.shape[0]
    cols = lax.broadcasted_iota(jnp.int32, (bm, B), 1)
    m = jnp.where(b_ref[...] == cols, 1.0, 0.0)
    s = lax.dot_general(m, h_ref[...], (((0,), (0,)), ((), ())),
                        preferred_element_type=jnp.float32)

    @pl.when(i == 0)
    def _():
        o_ref[...] = jnp.zeros_like(o_ref)
        c_ref[...] = jnp.zeros_like(c_ref)

    o_ref[...] += s
    c_ref[...] += jnp.sum(m, axis=0, keepdims=True)


def _segsum(h, batch2d, bm=2048):
    n = h.shape[0]
    return pl.pallas_call(
        _segsum_body,
        grid=(n // bm,),
        in_specs=[pl.BlockSpec((bm, EMB), lambda i: (i, 0)),
                  pl.BlockSpec((bm, 1), lambda i: (i, 0))],
        out_specs=[pl.BlockSpec((B, EMB), lambda i: (0, 0)),
                   pl.BlockSpec((1, B), lambda i: (0, 0))],
        out_shape=[jax.ShapeDtypeStruct((B, EMB), jnp.float32),
                   jax.ShapeDtypeStruct((1, B), jnp.float32)],
    )(h, batch2d)


def _rownorm_body(x_ref, o_ref):
    x = x_ref[...]
    nrm = jnp.sqrt(jnp.sum(x * x, axis=1, keepdims=True))
    o_ref[...] = x / jnp.maximum(nrm, 1e-12)


def _rownorm(x):
    return pl.pallas_call(
        _rownorm_body,
        grid=(1,),
        in_specs=[pl.BlockSpec((B, EMB), lambda i: (0, 0))],
        out_specs=pl.BlockSpec((B, EMB), lambda i: (0, 0)),
        out_shape=jax.ShapeDtypeStruct((B, EMB), jnp.float32),
    )(x)


_NROW = 128


def _simclr_body(reps_ref, a_ref, p_ref, o_ref):
    i = pl.program_id(0)
    n2 = 2 * B
    a = a_ref[...]
    s = lax.dot_general(a, reps_ref[...], (((1,), (1,)), ((), ())),
                        preferred_element_type=jnp.float32) / TEMP
    rows = lax.broadcasted_iota(jnp.int32, (_NROW, n2), 0) + i * _NROW
    cols = lax.broadcasted_iota(jnp.int32, (_NROW, n2), 1)
    s = jnp.where(rows == cols, -1e30, s)
    m = jnp.max(s, axis=1, keepdims=True)
    lse = m + jnp.log(jnp.sum(jnp.exp(s - m), axis=1, keepdims=True))
    d = jnp.sum(a * p_ref[...], axis=1, keepdims=True) / TEMP

    @pl.when(i == 0)
    def _():
        o_ref[...] = jnp.zeros_like(o_ref)

    contrib = jnp.sum(lse - d, axis=0, keepdims=True) / n2
    o_ref[...] += contrib


def _simclr(repsn):
    nblk = 2 * B // _NROW
    return pl.pallas_call(
        _simclr_body,
        grid=(nblk,),
        in_specs=[pl.BlockSpec((2 * B, EMB), lambda i: (0, 0)),
                  pl.BlockSpec((_NROW, EMB), lambda i: (i, 0)),
                  pl.BlockSpec((_NROW, EMB),
                               lambda i: (lax.rem(i + nblk // 2, nblk), 0))],
        out_specs=pl.BlockSpec((1, 1), lambda i: (0, 0)),
        out_shape=jax.ShapeDtypeStruct((1, 1), jnp.float32),
    )(repsn, repsn, repsn)


def _mlp_body(seg_ref, cnt_ref, dg_ref, pr_ref, pg_ref,
              w1_ref, b1_ref, w2_ref, b2_ref, o_ref):
    rep = seg_ref[...] / jnp.maximum(cnt_ref[...], 1.0)
    hid = (jnp.dot(rep, w1_ref[pl.ds(0, EMB), :],
                   preferred_element_type=jnp.float32)
           + jnp.dot(dg_ref[...], w1_ref[pl.ds(EMB, EMB), :],
                     preferred_element_type=jnp.float32)
           + jnp.dot(pr_ref[...], w1_ref[pl.ds(2 * EMB, EMB), :],
                     preferred_element_type=jnp.float32)
           + jnp.dot(pg_ref[...], w1_ref[pl.ds(3 * EMB, EMB), :],
                     preferred_element_type=jnp.float32)
           + b1_ref[...])
    hid = _leaky(hid)
    o_ref[...] = lax.dot_general(hid, w2_ref[...], (((1,), (0,)), ((), ())),
                                 preferred_element_type=jnp.float32) + b2_ref[...]


def _mlp(seg, cnt, dg, pr, pg, w1, b1, w2, b2):
    return pl.pallas_call(
        _mlp_body,
        grid=(1,),
        in_specs=[pl.BlockSpec((B, EMB), lambda i: (0, 0)),
                  pl.BlockSpec((B, 1), lambda i: (0, 0)),
                  pl.BlockSpec((B, EMB), lambda i: (0, 0)),
                  pl.BlockSpec((B, EMB), lambda i: (0, 0)),
                  pl.BlockSpec((B, EMB), lambda i: (0, 0)),
                  pl.BlockSpec((4 * EMB, EMB), lambda i: (0, 0)),
                  pl.BlockSpec((1, EMB), lambda i: (0, 0)),
                  pl.BlockSpec((EMB, 1), lambda i: (0, 0)),
                  pl.BlockSpec((1, 1), lambda i: (0, 0))],
        out_specs=pl.BlockSpec((B, 1), lambda i: (0, 0)),
        out_shape=jax.ShapeDtypeStruct((B, 1), jnp.float32),
    )(seg, cnt, dg, pr, pg, w1, b1, w2, b2)


def _sc_msg_build(n_table, n_edges, nout, scale, npass):
    nq = nout if npass == 1 else nout // (_NC * npass)
    per_tile = n_edges // (_NC * _NS) if npass == 1 else n_edges // _NS
    steps = per_tile // _CH
    acc_rows = max(2048, -(-(nq + npass - 1) // 2048) * 2048)
    zero_copies = acc_rows // (_NS * _CH)
    out_per = nq // _NS
    mesh = plsc.VectorSubcoreMesh(core_axis_name="c", subcore_axis_name="s")
    n_out_maj = _NC * npass

    @functools.partial(
        pl.kernel, mesh=mesh,
        out_type=jax.ShapeDtypeStruct((n_out_maj, nq, EMB), jnp.float32),
        scratch_types=[
            pltpu.VMEM((_CH,), jnp.int32),
            pltpu.VMEM((_CH,), jnp.int32),
            pltpu.VMEM((1, _CH), jnp.int32),
            pltpu.VMEM((_CH,), jnp.float32),
            pltpu.VMEM((_CH, EMB), jnp.float32),
            pltpu.VMEM_SHARED((acc_rows, EMB), jnp.float32),
            pltpu.SemaphoreType.DMA,
        ])
    def k(table_h, src_h, dst_h, ew_h, out_h,
          sidx, dstst, didx, ewv, rows, acc, sem):
        c = lax.axis_index("c")
        s = lax.axis_index("s")

        def zero_rows(r, _):
            for j in range(EMB // 16):
                rows[r, pl.ds(j * 16, 16)] = jnp.zeros((16,), jnp.float32)
            return 0

        lax.fori_loop(0, _CH, zero_rows, 0)

        for p in range(npass):
            for t in range(zero_copies):
                pltpu.sync_copy(
                    rows, acc.at[pl.ds(s * (zero_copies * _CH) + t * _CH, _CH)])
            plsc.subcore_barrier()

            if npass == 1:
                base = (c * _NS + s) * per_tile
            else:
                base = s * per_tile
            qbase = (c * npass + p) * nq

            def step(t, _):
                off = base + t * _CH
                pltpu.sync_copy(src_h.at[pl.ds(off, _CH)], sidx)
                pltpu.async_copy(table_h.at[sidx], rows, sem).wait()
                if npass == 1:
                    pltpu.sync_copy(dst_h.at[pl.ds(off, _CH)], didx.at[0])
                else:
                    pltpu.sync_copy(dst_h.at[pl.ds(off, _CH)], dstst)
                    for kk in range(_CH // 16):
                        d16 = dstst[pl.ds(kk * 16, 16)] - qbase
                        ok = (d16 >= 0) & (d16 < nq)
                        didx[0, pl.ds(kk * 16, 16)] = jnp.where(ok, d16, nq)
                if scale:
                    pltpu.sync_copy(ew_h.at[pl.ds(off, _CH)], ewv)

                    def scale_grp(g, _):
                        wv = ewv[pl.ds(g * 16, 16)]
                        for l in range(16):
                            e = g * 16 + l
                            w = wv[l]
                            for j in range(EMB // 16):
                                rows[e, pl.ds(j * 16, 16)] = (
                                    rows[e, pl.ds(j * 16, 16)] * w)
                        return 0

                    lax.fori_loop(0, _CH // 16, scale_grp, 0)
                pltpu.sync_copy(rows, acc.at[didx.at[0]], add=True)
                return 0

            lax.fori_loop(0, steps, step, 0)
            plsc.subcore_barrier()

            q = c * npass + p if npass > 1 else c
            pltpu.sync_copy(acc.at[pl.ds(s * out_per, out_per)],
                            out_h.at[q, pl.ds(s * out_per, out_per)])
            if npass > 1 and p + 1 < npass:
                plsc.subcore_barrier()
            if p + 1 < npass:
                lax.fori_loop(0, _CH, zero_rows, 0)

    return k


def _sc_msg(table, src, dst, ew, nout, scale, npass):
    k = _sc_msg_build(table.shape[0], src.shape[0], nout, scale, npass)
    return k(table, src, dst, ew)


def _scatter_rows(hs, src, dst, ew, nout):
    return jnp.zeros((nout, EMB), jnp.float32).at[dst].add(
        ew[:, None] * hs[src])


def _hist(idx, w, nout):
    return jnp.zeros((nout,), jnp.float32).at[idx].add(w)


def kernel(drug_x, edge_index, batch, protein_seq, drug_ids, prot_ids,
           affinity_x, affinity_edge_index, affinity_adj,
           Wd1, bd1, Wd2, bd2, emb, cw1, cb1, cw2, cb2,
           Wg1, bg1, Wg2, bg2, pw1, pb1, pw2, pb2):
    src = edge_index[0]
    dst = edge_index[1]
    gsrc = affinity_edge_index[0]
    gdst = affinity_edge_index[1]

    ew_dummy = jnp.zeros((_CH,), jnp.float32)
    degd = _hist(dst, jnp.ones_like(dst, jnp.float32), N_ATOM)[:, None]
    hs1 = _mm_scale(drug_x, Wd1, degd)
    msg1 = _sc_msg(hs1, src, dst, ew_dummy, N_ATOM, False, 2)
    msg1 = msg1.reshape(N_ATOM, EMB)
    hs2 = _combine_mm(msg1, hs1, degd, bd1[None, :], Wd2)
    msg2 = _sc_msg(hs2, src, dst, ew_dummy, N_ATOM, False, 2)
    msg2 = msg2.reshape(N_ATOM, EMB)
    g2d = _combine(msg2, hs2, degd, bd2[None, :])
    segsum, cnt = _segsum(g2d, batch[:, None])
    cnt_col = cnt.reshape(B, 1)

    emb_pad = jnp.concatenate([emb, jnp.zeros((6, EMB), jnp.float32)], axis=0)
    w1s = jnp.stack([cw1[:, :, k].T for k in range(3)], axis=0)
    w2s = jnp.stack([cw2[:, :, k].T for k in range(3)], axis=0)
    prot_rep = _prot_head(protein_seq.reshape(-1, 1), emb_pad,
                          w1s, cb1[None, :], w2s, cb2[None, :])

    ew = affinity_adj[gsrc, gdst]
    dega = _hist(gdst, ew, NAFF_PAD)[:, None]
    ax_pad = jnp.concatenate(
        [affinity_x, jnp.zeros((NAFF_PAD - N_AFF, EMB), jnp.float32)], axis=0)
    ga1 = _mm_scale(ax_pad, Wg1, dega)
    epad = 655360 - gsrc.shape[0]
    gsrcp = jnp.concatenate([gsrc, jnp.zeros((epad,), gsrc.dtype)])
    gdstp = jnp.concatenate([gdst, jnp.zeros((epad,), gdst.dtype)])
    ewp = jnp.concatenate([ew, jnp.zeros((epad,), jnp.float32)])
    ma1 = _sc_msg(ga1, gsrcp, gdstp, ewp, NAFF_PAD, True, 1)
    ga2 = _combine_mm(ma1[0] + ma1[1], ga1, dega, bg1[None, :], Wg2)
    ma2 = _sc_msg(ga2, gsrcp, gdstp, ewp, NAFF_PAD, True, 1)
    g2a = _combine(ma2[0] + ma2[1], ga2, dega, bg2[None, :])

    drug_g = g2a[drug_ids]
    prot_g = g2a[prot_ids + NUM_DRUG]

    z1n = _rownorm(segsum)
    z2n = _rownorm(drug_g)
    loss_d = _simclr(jnp.concatenate([z1n, z2n], axis=0))[0, 0]
    p1n = _rownorm(prot_rep)
    p2n = _rownorm(prot_g)
    loss_p = _simclr(jnp.concatenate([p1n, p2n], axis=0))[0, 0]

    y = _mlp(segsum, cnt_col, drug_g, prot_rep, prot_g,
             pw1, pb1[None, :], pw2, pb2[None, :])
    return (y[:, 0], loss_d, loss_p)


# speedup vs baseline: 6.7790x; 1.2989x over previous
import functools

import jax
import jax.numpy as jnp
from jax import lax
from jax.experimental import pallas as pl
from jax.experimental.pallas import tpu as pltpu
from jax.experimental.pallas import tpu_sc as plsc

_NC = 2
_NS = 16
_CH = 128

EMB = 128
B = 1024
TARGET_LEN = 128
N_ATOM = 32768
NUM_DRUG = 2000
N_AFF = 10000
NAFF_PAD = 10240
TEMP = 0.5


def _leaky(x):
    return jnp.where(x >= 0, x, 0.01 * x)


def _mm_scale_body(x_ref, w_ref, deg_ref, o_ref):
    dinv = lax.rsqrt(deg_ref[...] + 1.0)
    o_ref[...] = dinv * jnp.dot(x_ref[...], w_ref[...],
                                preferred_element_type=jnp.float32)


def _mm_scale(x, w, deg, bm=2048):
    n = x.shape[0]
    return pl.pallas_call(
        _mm_scale_body,
        grid=(n // bm,),
        in_specs=[pl.BlockSpec((bm, EMB), lambda i: (i, 0)),
                  pl.BlockSpec((EMB, EMB), lambda i: (0, 0)),
                  pl.BlockSpec((bm, 1), lambda i: (i, 0))],
        out_specs=pl.BlockSpec((bm, EMB), lambda i: (i, 0)),
        out_shape=jax.ShapeDtypeStruct((n, EMB), jnp.float32),
    )(x, w, deg)


def _combine_mm_body(msg_ref, hs_ref, deg_ref, b_ref, w_ref, o_ref):
    dinv = lax.rsqrt(deg_ref[...] + 1.0)
    g = _leaky(dinv * (msg_ref[...] + hs_ref[...]) + b_ref[...])
    o_ref[...] = dinv * jnp.dot(g, w_ref[...],
                                preferred_element_type=jnp.float32)


def _combine_mm(msg, hs, deg, b, w, bm=2048):
    n = msg.shape[0]
    return pl.pallas_call(
        _combine_mm_body,
        grid=(n // bm,),
        in_specs=[pl.BlockSpec((bm, EMB), lambda i: (i, 0)),
                  pl.BlockSpec((bm, EMB), lambda i: (i, 0)),
                  pl.BlockSpec((bm, 1), lambda i: (i, 0)),
                  pl.BlockSpec((1, EMB), lambda i: (0, 0)),
                  pl.BlockSpec((EMB, EMB), lambda i: (0, 0))],
        out_specs=pl.BlockSpec((bm, EMB), lambda i: (i, 0)),
        out_shape=jax.ShapeDtypeStruct((n, EMB), jnp.float32),
    )(msg, hs, deg, b, w)


def _combine_body(msg_ref, hs_ref, deg_ref, b_ref, o_ref):
    dinv = lax.rsqrt(deg_ref[...] + 1.0)
    o_ref[...] = _leaky(dinv * (msg_ref[...] + hs_ref[...]) + b_ref[...])


def _combine(msg, hs, deg, b, bm=2048):
    n = msg.shape[0]
    return pl.pallas_call(
        _combine_body,
        grid=(n // bm,),
        in_specs=[pl.BlockSpec((bm, EMB), lambda i: (i, 0)),
                  pl.BlockSpec((bm, EMB), lambda i: (i, 0)),
                  pl.BlockSpec((bm, 1), lambda i: (i, 0)),
                  pl.BlockSpec((1, EMB), lambda i: (0, 0))],
        out_specs=pl.BlockSpec((bm, EMB), lambda i: (i, 0)),
        out_shape=jax.ShapeDtypeStruct((n, EMB), jnp.float32),
    )(msg, hs, deg, b)


_NB = 32


def _prot_body(seq_ref, emb_ref, w1_ref, b1_ref, w2_ref, b2_ref, o_ref):
    R = _NB * TARGET_LEN
    seq = seq_ref[...]
    cols = lax.broadcasted_iota(jnp.int32, (R, 32), 1)
    onehot = jnp.where(seq == cols, 1.0, 0.0)
    x = jnp.dot(onehot, emb_ref[...], preferred_element_type=jnp.float32)
    pos = lax.broadcasted_iota(jnp.int32, (R, 1), 0) % TARGET_LEN
    zrow = jnp.zeros((1, EMB), jnp.float32)

    def block(xx, w_ref, b_ref):
        down = jnp.concatenate([zrow, xx[:-1, :]], axis=0)
        up = jnp.concatenate([xx[1:, :], zrow], axis=0)
        down = jnp.where(pos == 0, 0.0, down)
        up = jnp.where(pos == TARGET_LEN - 1, 0.0, up)
        y = (jnp.dot(down, w_ref[0], preferred_element_type=jnp.float32)
             + jnp.dot(xx, w_ref[1], preferred_element_type=jnp.float32)
             + jnp.dot(up, w_ref[2], preferred_element_type=jnp.float32)
             + b_ref[...])
        y = y * (1.0 / jnp.sqrt(1.0 + 1e-05))
        return _leaky(y) + xx

    x = block(x, w1_ref, b1_ref)
    x = block(x, w2_ref, b2_ref)
    rr = lax.broadcasted_iota(jnp.int32, (_NB, R), 1)
    bb = lax.broadcasted_iota(jnp.int32, (_NB, R), 0)
    sel = jnp.where(rr // TARGET_LEN == bb, 1.0 / TARGET_LEN, 0.0)
    o_ref[...] = jnp.dot(sel, x, preferred_element_type=jnp.float32)


def _prot_head(seq_flat, emb_pad, w1s, b1, w2s, b2):
    R = _NB * TARGET_LEN
    return pl.pallas_call(
        _prot_body,
        grid=(B // _NB,),
        in_specs=[pl.BlockSpec((R, 1), lambda i: (i, 0)),
                  pl.BlockSpec((32, EMB), lambda i: (0, 0)),
                  pl.BlockSpec((3, EMB, EMB), lambda i: (0, 0, 0)),
                  pl.BlockSpec((1, EMB), lambda i: (0, 0)),
                  pl.BlockSpec((3, EMB, EMB), lambda i: (0, 0, 0)),
                  pl.BlockSpec((1, EMB), lambda i: (0, 0))],
        out_specs=pl.BlockSpec((_NB, EMB), lambda i: (i, 0)),
        out_shape=jax.ShapeDtypeStruct((B, EMB), jnp.float32),
    )(seq_flat, emb_pad, w1s, b1, w2s, b2)


def _segsum_body(h_ref, b_ref, o_ref, c_ref):
    i = pl.program_id(0)
    bm = h_ref.shape[0]
    cols = lax.broadcasted_iota(jnp.int32, (bm, B), 1)
    m = jnp.where(b_ref[...] == cols, 1.0, 0.0)
    s = lax.dot_general(m, h_ref[...], (((0,), (0,)), ((), ())),
                        preferred_element_type=jnp.float32)

    @pl.when(i == 0)
    def _():
        o_ref[...] = jnp.zeros_like(o_ref)
        c_ref[...] = jnp.zeros_like(c_ref)

    o_ref[...] += s
    c_ref[...] += jnp.sum(m, axis=0, keepdims=True)


def _segsum(h, batch2d, bm=2048):
    n = h.shape[0]
    return pl.pallas_call(
        _segsum_body,
        grid=(n // bm,),
        in_specs=[pl.BlockSpec((bm, EMB), lambda i: (i, 0)),
                  pl.BlockSpec((bm, 1), lambda i: (i, 0))],
        out_specs=[pl.BlockSpec((B, EMB), lambda i: (0, 0)),
                   pl.BlockSpec((1, B), lambda i: (0, 0))],
        out_shape=[jax.ShapeDtypeStruct((B, EMB), jnp.float32),
                   jax.ShapeDtypeStruct((1, B), jnp.float32)],
    )(h, batch2d)


def _rownorm_body(x_ref, o_ref):
    x = x_ref[...]
    nrm = jnp.sqrt(jnp.sum(x * x, axis=1, keepdims=True))
    o_ref[...] = x / jnp.maximum(nrm, 1e-12)


def _rownorm(x):
    return pl.pallas_call(
        _rownorm_body,
        grid=(1,),
        in_specs=[pl.BlockSpec((B, EMB), lambda i: (0, 0))],
        out_specs=pl.BlockSpec((B, EMB), lambda i: (0, 0)),
        out_shape=jax.ShapeDtypeStruct((B, EMB), jnp.float32),
    )(x)


_NROW = 128


def _simclr_body(reps_ref, a_ref, p_ref, o_ref):
    i = pl.program_id(0)
    n2 = 2 * B
    a = a_ref[...]
    s = lax.dot_general(a, reps_ref[...], (((1,), (1,)), ((), ())),
                        preferred_element_type=jnp.float32) / TEMP
    rows = lax.broadcasted_iota(jnp.int32, (_NROW, n2), 0) + i * _NROW
    cols = lax.broadcasted_iota(jnp.int32, (_NROW, n2), 1)
    s = jnp.where(rows == cols, -1e30, s)
    m = jnp.max(s, axis=1, keepdims=True)
    lse = m + jnp.log(jnp.sum(jnp.exp(s - m), axis=1, keepdims=True))
    d = jnp.sum(a * p_ref[...], axis=1, keepdims=True) / TEMP

    @pl.when(i == 0)
    def _():
        o_ref[...] = jnp.zeros_like(o_ref)

    contrib = jnp.sum(lse - d, axis=0, keepdims=True) / n2
    o_ref[...] += contrib


def _simclr(repsn):
    nblk = 2 * B // _NROW
    return pl.pallas_call(
        _simclr_body,
        grid=(nblk,),
        in_specs=[pl.BlockSpec((2 * B, EMB), lambda i: (0, 0)),
                  pl.BlockSpec((_NROW, EMB), lambda i: (i, 0)),
                  pl.BlockSpec((_NROW, EMB),
                               lambda i: (lax.rem(i + nblk // 2, nblk), 0))],
        out_specs=pl.BlockSpec((1, 1), lambda i: (0, 0)),
        out_shape=jax.ShapeDtypeStruct((1, 1), jnp.float32),
    )(repsn, repsn, repsn)


def _mlp_body(seg_ref, cnt_ref, dg_ref, pr_ref, pg_ref,
              w1_ref, b1_ref, w2_ref, b2_ref, o_ref):
    rep = seg_ref[...] / jnp.maximum(cnt_ref[...], 1.0)
    hid = (jnp.dot(rep, w1_ref[pl.ds(0, EMB), :],
                   preferred_element_type=jnp.float32)
           + jnp.dot(dg_ref[...], w1_ref[pl.ds(EMB, EMB), :],
                     preferred_element_type=jnp.float32)
           + jnp.dot(pr_ref[...], w1_ref[pl.ds(2 * EMB, EMB), :],
                     preferred_element_type=jnp.float32)
           + jnp.dot(pg_ref[...], w1_ref[pl.ds(3 * EMB, EMB), :],
                     preferred_element_type=jnp.float32)
           + b1_ref[...])
    hid = _leaky(hid)
    o_ref[...] = lax.dot_general(hid, w2_ref[...], (((1,), (0,)), ((), ())),
                                 preferred_element_type=jnp.float32) + b2_ref[...]


def _mlp(seg, cnt, dg, pr, pg, w1, b1, w2, b2):
    return pl.pallas_call(
        _mlp_body,
        grid=(1,),
        in_specs=[pl.BlockSpec((B, EMB), lambda i: (0, 0)),
                  pl.BlockSpec((B, 1), lambda i: (0, 0)),
                  pl.BlockSpec((B, EMB), lambda i: (0, 0)),
                  pl.BlockSpec((B, EMB), lambda i: (0, 0)),
                  pl.BlockSpec((B, EMB), lambda i: (0, 0)),
                  pl.BlockSpec((4 * EMB, EMB), lambda i: (0, 0)),
                  pl.BlockSpec((1, EMB), lambda i: (0, 0)),
                  pl.BlockSpec((EMB, 1), lambda i: (0, 0)),
                  pl.BlockSpec((1, 1), lambda i: (0, 0))],
        out_specs=pl.BlockSpec((B, 1), lambda i: (0, 0)),
        out_shape=jax.ShapeDtypeStruct((B, 1), jnp.float32),
    )(seg, cnt, dg, pr, pg, w1, b1, w2, b2)


def _sc_msg_build(n_table, n_edges, nout, scale, npass):
    nq = nout if npass == 1 else nout // (_NC * npass)
    per_tile = n_edges // (_NC * _NS) if npass == 1 else n_edges // _NS
    steps = per_tile // _CH
    acc_rows = max(2048, -(-(nq + npass - 1) // 2048) * 2048)
    zero_copies = acc_rows // (_NS * _CH)
    out_per = nq // _NS
    mesh = plsc.VectorSubcoreMesh(core_axis_name="c", subcore_axis_name="s")
    n_out_maj = _NC * npass
    pw = 2

    @functools.partial(
        pl.kernel, mesh=mesh,
        out_type=jax.ShapeDtypeStruct((n_out_maj, nq, EMB), jnp.float32),
        scratch_types=[
            pltpu.VMEM((pw * _CH,), jnp.int32),
            pltpu.VMEM((pw * _CH,), jnp.int32),
            pltpu.VMEM((_CH,), jnp.int32),
            pltpu.VMEM((_CH,), jnp.int32),
            pltpu.VMEM((1, _CH), jnp.int32),
            pltpu.VMEM((1, _CH), jnp.int32),
            pltpu.VMEM((_CH,), jnp.float32),
            pltpu.VMEM((_CH,), jnp.float32),
            pltpu.VMEM((_CH, EMB), jnp.float32),
            pltpu.VMEM((_CH, EMB), jnp.float32),
            pltpu.VMEM_SHARED((acc_rows, EMB), jnp.float32),
            pltpu.SemaphoreType.DMA,
            pltpu.SemaphoreType.DMA,
        ])
    def k(table_h, pack_h, ew_h, out_h,
          pk0, pk1, si0, si1, dd0, dd1, ew0, ew1, rows0, rows1, acc,
          gsem0, gsem1):
        pks = (pk0, pk1)
        sis = (si0, si1)
        dds = (dd0, dd1)
        ews = (ew0, ew1)
        rowss = (rows0, rows1)
        gsems = (gsem0, gsem1)
        c = lax.axis_index("c")
        s = lax.axis_index("s")

        def zero_rows(r, _):
            for j in range(EMB // 16):
                rows0[r, pl.ds(j * 16, 16)] = jnp.zeros((16,), jnp.float32)
            return 0

        lax.fori_loop(0, _CH, zero_rows, 0)

        for p in range(npass):
            for t in range(zero_copies):
                pltpu.sync_copy(
                    rows0,
                    acc.at[pl.ds(s * (zero_copies * _CH) + t * _CH, _CH)])
            plsc.subcore_barrier()

            if npass == 1:
                cbase = (c * _NS + s) * steps
            else:
                cbase = s * steps
            qbase = (c * npass + p) * nq

            def load_idx(t, b):
                tc = cbase + jnp.minimum(t, steps - 1)
                pltpu.sync_copy(pack_h.at[pl.ds(tc * (pw * _CH), pw * _CH)],
                                pks[b])
                if scale:
                    pltpu.sync_copy(ew_h.at[pl.ds(tc * _CH, _CH)], ews[b])
                for kk in range(_CH // 16):
                    sis[b][pl.ds(kk * 16, 16)] = pks[b][pl.ds(kk * 16, 16)]

            def issue_gather(b):
                pltpu.async_copy(table_h.at[sis[b]], rowss[b], gsems[b])

            def wait_gather(b):
                pltpu.make_async_copy(table_h.at[sis[b]], rowss[b],
                                      gsems[b]).wait()

            load_idx(0, 0)
            issue_gather(0)

            def half(t, b):
                nb = 1 - b
                rows = rowss[b]
                pk = pks[b]
                wait_gather(b)
                load_idx(t + 1, nb)
                issue_gather(nb)
                if npass > 1:
                    for kk in range(_CH // 16):
                        d16 = pk[pl.ds(_CH + kk * 16, 16)] - qbase
                        ok = (d16 >= 0) & (d16 < nq)
                        dds[b][0, pl.ds(kk * 16, 16)] = jnp.where(ok, d16, nq)
                else:
                    for kk in range(_CH // 16):
                        dds[b][0, pl.ds(kk * 16, 16)] = (
                            pk[pl.ds(_CH + kk * 16, 16)])
                if scale:
                    def scale_grp(g, _):
                        wv = ews[b][pl.ds(g * 16, 16)]
                        for l in range(16):
                            e = g * 16 + l
                            w = wv[l]
                            for j in range(EMB // 16):
                                rows[e, pl.ds(j * 16, 16)] = (
                                    rows[e, pl.ds(j * 16, 16)] * w)
                        return 0

                    lax.fori_loop(0, _CH // 16, scale_grp, 0)
                pltpu.sync_copy(rows, acc.at[dds[b].at[0]], add=True)

            def step2(g, _):
                half(2 * g, 0)
                half(2 * g + 1, 1)
                return 0

            lax.fori_loop(0, steps // 2, step2, 0)
            wait_gather(0)
            plsc.subcore_barrier()

            q = c * npass + p if npass > 1 else c
            pltpu.sync_copy(acc.at[pl.ds(s * out_per, out_per)],
                            out_h.at[q, pl.ds(s * out_per, out_per)])
            if npass > 1 and p + 1 < npass:
                plsc.subcore_barrier()
            if p + 1 < npass:
                lax.fori_loop(0, _CH, zero_rows, 0)

    return k


def _pack_edges(src, dst, ew=None):
    m = src.shape[0] // _CH
    parts = [src.reshape(m, 1, _CH), dst.reshape(m, 1, _CH)]
    if ew is not None:
        parts.append(
            lax.bitcast_convert_type(ew, jnp.int32).reshape(m, 1, _CH))
    return jnp.concatenate(parts, axis=1).reshape(-1)


def _sc_msg(table, pack, ew, nout, scale, npass):
    n_edges = pack.shape[0] // 2
    k = _sc_msg_build(table.shape[0], n_edges, nout, scale, npass)
    return k(table, pack, ew)


def _scatter_rows(hs, src, dst, ew, nout):
    return jnp.zeros((nout, EMB), jnp.float32).at[dst].add(
        ew[:, None] * hs[src])


def _hist(idx, w, nout):
    return jnp.zeros((nout,), jnp.float32).at[idx].add(w)


def kernel(drug_x, edge_index, batch, protein_seq, drug_ids, prot_ids,
           affinity_x, affinity_edge_index, affinity_adj,
           Wd1, bd1, Wd2, bd2, emb, cw1, cb1, cw2, cb2,
           Wg1, bg1, Wg2, bg2, pw1, pb1, pw2, pb2):
    src = edge_index[0]
    dst = edge_index[1]
    gsrc = affinity_edge_index[0]
    gdst = affinity_edge_index[1]

    packd = _pack_edges(src, dst)
    degd = _hist(dst, jnp.ones_like(dst, jnp.float32), N_ATOM)[:, None]
    hs1 = _mm_scale(drug_x, Wd1, degd)
    ew_dummy = jnp.zeros((_CH,), jnp.float32)
    msg1 = _sc_msg(hs1, packd, ew_dummy, N_ATOM, False, 2).reshape(N_ATOM, EMB)
    hs2 = _combine_mm(msg1, hs1, degd, bd1[None, :], Wd2)
    msg2 = _sc_msg(hs2, packd, ew_dummy, N_ATOM, False, 2).reshape(N_ATOM, EMB)
    g2d = _combine(msg2, hs2, degd, bd2[None, :])
    segsum, cnt = _segsum(g2d, batch[:, None])
    cnt_col = cnt.reshape(B, 1)

    emb_pad = jnp.concatenate([emb, jnp.zeros((6, EMB), jnp.float32)], axis=0)
    w1s = jnp.stack([cw1[:, :, k].T for k in range(3)], axis=0)
    w2s = jnp.stack([cw2[:, :, k].T for k in range(3)], axis=0)
    prot_rep = _prot_head(protein_seq.reshape(-1, 1), emb_pad,
                          w1s, cb1[None, :], w2s, cb2[None, :])

    ew = affinity_adj[gsrc, gdst]
    dega = _hist(gdst, ew, NAFF_PAD)[:, None]
    ax_pad = jnp.concatenate(
        [affinity_x, jnp.zeros((NAFF_PAD - N_AFF, EMB), jnp.float32)], axis=0)
    ga1 = _mm_scale(ax_pad, Wg1, dega)
    epad = 655360 - gsrc.shape[0]
    gsrcp = jnp.concatenate([gsrc, jnp.zeros((epad,), gsrc.dtype)])
    gdstp = jnp.concatenate([gdst, jnp.zeros((epad,), gdst.dtype)])
    ewp = jnp.concatenate([ew, jnp.zeros((epad,), jnp.float32)])
    packa = _pack_edges(gsrcp, gdstp)
    ma1 = _sc_msg(ga1, packa, ewp, NAFF_PAD, True, 1)
    ga2 = _combine_mm(ma1[0] + ma1[1], ga1, dega, bg1[None, :], Wg2)
    ma2 = _sc_msg(ga2, packa, ewp, NAFF_PAD, True, 1)
    g2a = _combine(ma2[0] + ma2[1], ga2, dega, bg2[None, :])

    drug_g = g2a[drug_ids]
    prot_g = g2a[prot_ids + NUM_DRUG]

    z1n = _rownorm(segsum)
    z2n = _rownorm(drug_g)
    loss_d = _simclr(jnp.concatenate([z1n, z2n], axis=0))[0, 0]
    p1n = _rownorm(prot_rep)
    p2n = _rownorm(prot_g)
    loss_p = _simclr(jnp.concatenate([p1n, p2n], axis=0))[0, 0]

    y = _mlp(segsum, cnt_col, drug_g, prot_rep, prot_g,
             pw1, pb1[None, :], pw2, pb2[None, :])
    return (y[:, 0], loss_d, loss_p)
